# Initial kernel scaffold; baseline (speedup 1.0000x reference)
import jax
import jax.numpy as jnp
from jax.experimental import pallas as pl


def kernel(x, edge_index, edge_attr, dW0, db0, W0, b0, dW1, db1, W1, b1):
    raise NotImplementedError("write your pallas kernel here")


# speedup vs baseline: 288.6453x; 288.6453x over previous
import functools

def _z(i):
    return i * 0

import jax
import jax.numpy as jnp
from jax import lax
from jax.experimental import pallas as pl
from jax.experimental.pallas import tpu as pltpu
from jax.experimental.pallas import tpu_sc as plsc

N = 10000
E = 160000
WD = 32
EF = 16

NC = 2
NS = 16
NW = NC * NS
EPW = E // NW
CHUNK = 1000
NCHUNK = EPW // CHUNK
RPT = N // NS

def _gather_body(x_hbm, src_hbm, out_hbm, idx_v, rows_v, sem):
    wid = lax.axis_index("c") * NS + lax.axis_index("s")
    base = wid * EPW
    for c in range(NCHUNK):
        off = base + c * CHUNK
        pltpu.sync_copy(src_hbm.at[pl.ds(off, CHUNK)], idx_v)
        pltpu.async_copy(x_hbm.at[idx_v], rows_v, sem).wait()
        pltpu.sync_copy(rows_v, out_hbm.at[pl.ds(off, CHUNK)])


@functools.lru_cache(maxsize=None)
def _sc_kernels():
    mesh = plsc.VectorSubcoreMesh(core_axis_name="c", subcore_axis_name="s")
    cp = pltpu.CompilerParams(use_tc_tiling_on_sc=False)
    gather = pl.kernel(
        _gather_body,
        mesh=mesh,
        compiler_params=cp,
        out_type=jax.ShapeDtypeStruct((E, WD), jnp.float32),
        scratch_types=[
            pltpu.VMEM((CHUNK,), jnp.int32),
            pltpu.VMEM((CHUNK, WD), jnp.float32),
            pltpu.SemaphoreType.DMA,
        ],
    )
    scatter_cnt = pl.kernel(
        functools.partial(_scatter_body, True),
        mesh=mesh,
        compiler_params=cp,
        out_type=(
            jax.ShapeDtypeStruct((NC * N, WD), jnp.float32),
            jax.ShapeDtypeStruct((NC * N, EF), jnp.float32),
        ),
        scratch_types=[
            pltpu.VMEM((CHUNK,), jnp.int32),
            pltpu.VMEM((CHUNK, WD), jnp.float32),
            pltpu.VMEM((CHUNK, EF), jnp.float32),
            pltpu.VMEM_SHARED((N, WD), jnp.float32),
            pltpu.VMEM_SHARED((N, EF), jnp.float32),
        ],
    )
    scatter = pl.kernel(
        functools.partial(_scatter_body, False),
        mesh=mesh,
        compiler_params=cp,
        out_type=jax.ShapeDtypeStruct((NC * N, WD), jnp.float32),
        scratch_types=[
            pltpu.VMEM((CHUNK,), jnp.int32),
            pltpu.VMEM((CHUNK, WD), jnp.float32),
            pltpu.VMEM_SHARED((N, WD), jnp.float32),
        ],
    )
    return gather, scatter_cnt, scatter


def _scatter_body(with_cnt, msg_hbm, dst_hbm, z32_hbm, *rest):
    if with_cnt:
        (z16_hbm, ones_hbm, agg_hbm, cnt_hbm,
         idx_v, rows_v, ones_v, agg_sh, cnt_sh) = rest
    else:
        agg_hbm, idx_v, rows_v, agg_sh = rest
    cid = lax.axis_index("c")
    sid = lax.axis_index("s")
    wid = cid * NS + sid
    pltpu.sync_copy(z32_hbm.at[pl.ds(sid * RPT, RPT)],
                    agg_sh.at[pl.ds(sid * RPT, RPT)])
    if with_cnt:
        pltpu.sync_copy(z16_hbm.at[pl.ds(sid * RPT, RPT)],
                        cnt_sh.at[pl.ds(sid * RPT, RPT)])
        pltpu.sync_copy(ones_hbm, ones_v)
    plsc.subcore_barrier()
    base = wid * EPW
    for c in range(NCHUNK):
        off = base + c * CHUNK
        pltpu.sync_copy(dst_hbm.at[pl.ds(off, CHUNK)], idx_v)
        pltpu.sync_copy(msg_hbm.at[pl.ds(off, CHUNK)], rows_v)
        pltpu.sync_copy(rows_v, agg_sh.at[idx_v], add=True)
        if with_cnt:
            pltpu.sync_copy(ones_v, cnt_sh.at[idx_v], add=True)
    plsc.subcore_barrier()
    pltpu.sync_copy(agg_sh.at[pl.ds(sid * RPT, RPT)],
                    agg_hbm.at[pl.ds(cid * N + sid * RPT, RPT)])
    if with_cnt:
        pltpu.sync_copy(cnt_sh.at[pl.ds(sid * RPT, RPT)],
                        cnt_hbm.at[pl.ds(cid * N + sid * RPT, RPT)])


BE = 2000


def _edge_body(xs_ref, ea_ref, dkt_ref, dbt_ref, msg_ref):
    xs = xs_ref[...]
    g = jnp.dot(xs, dkt_ref[...], preferred_element_type=jnp.float32,
                precision=lax.Precision.HIGHEST)
    acc = jnp.dot(xs, dbt_ref[...], preferred_element_type=jnp.float32,
                  precision=lax.Precision.HIGHEST)
    ea = ea_ref[...]
    for f in range(EF):
        acc = acc + ea[:, f:f + 1] * g[:, f * WD:(f + 1) * WD]
    msg_ref[...] = acc


def _tc_edge(xs, ea, dkt, dbt):
    return pl.pallas_call(
        _edge_body,
        grid=(E // BE,),
        in_specs=[
            pl.BlockSpec((BE, WD), lambda i: (i, _z(i))),
            pl.BlockSpec((BE, EF), lambda i: (i, _z(i))),
            pl.BlockSpec((WD, EF * WD), lambda i: (_z(i), _z(i))),
            pl.BlockSpec((WD, WD), lambda i: (_z(i), _z(i))),
        ],
        out_specs=pl.BlockSpec((BE, WD), lambda i: (i, _z(i))),
        out_shape=jax.ShapeDtypeStruct((E, WD), jnp.float32),
    )(xs, ea, dkt, dbt)


BN = 2000


def _fin_body(a0_ref, a1_ref, c0_ref, c1_ref, x_ref, wt_ref, b_ref, out_ref):
    agg = a0_ref[...] + a1_ref[...]
    cnt = c0_ref[:, 0:1] + c1_ref[:, 0:1]
    mean = agg / jnp.maximum(cnt, 1.0)
    lin = jnp.dot(x_ref[...], wt_ref[...], preferred_element_type=jnp.float32,
                  precision=lax.Precision.HIGHEST)
    out_ref[...] = jnp.tanh(mean + lin + b_ref[...])


def _tc_finalize(aggp, cntp, x, wt, b2):
    nb = N // BN
    return pl.pallas_call(
        _fin_body,
        grid=(nb,),
        in_specs=[
            pl.BlockSpec((BN, WD), lambda i: (i, _z(i))),
            pl.BlockSpec((BN, WD), lambda i: (i + nb, _z(i))),
            pl.BlockSpec((BN, EF), lambda i: (i, _z(i))),
            pl.BlockSpec((BN, EF), lambda i: (i + nb, _z(i))),
            pl.BlockSpec((BN, WD), lambda i: (i, _z(i))),
            pl.BlockSpec((WD, WD), lambda i: (_z(i), _z(i))),
            pl.BlockSpec((1, WD), lambda i: (_z(i), _z(i))),
        ],
        out_specs=pl.BlockSpec((BN, WD), lambda i: (i, _z(i))),
        out_shape=jax.ShapeDtypeStruct((N, WD), jnp.float32),
    )(aggp, aggp, cntp, cntp, x, wt, b2)


def kernel(x, edge_index, edge_attr, dW0, db0, W0, b0, dW1, db1, W1, b1):
    src = edge_index[0].astype(jnp.int32)
    dst = edge_index[1].astype(jnp.int32)
    ea = edge_attr.astype(jnp.float32)
    h = x.astype(jnp.float32)
    z32 = jnp.zeros((N, WD), jnp.float32)
    z16 = jnp.zeros((N, EF), jnp.float32)
    ones16 = jnp.ones((CHUNK, EF), jnp.float32)

    cntp = None
    for dW, db, W, b in ((dW0, db0, W0, b0), (dW1, db1, W1, b1)):
        dkt = jnp.transpose(dW.astype(jnp.float32).reshape(EF, WD, WD),
                            (2, 0, 1)).reshape(WD, EF * WD)
        dbt = db.astype(jnp.float32).reshape(WD, WD).T
        wt = W.astype(jnp.float32).T
        b2 = b.astype(jnp.float32).reshape(1, WD)

        sc_gather, sc_scatter_cnt, sc_scatter = _sc_kernels()
        xs = sc_gather(h, src)
        msg = _tc_edge(xs, ea, dkt, dbt)
        if cntp is None:
            aggp, cntp = sc_scatter_cnt(msg, dst, z32, z16, ones16)
        else:
            aggp = sc_scatter(msg, dst, z32)
        h = _tc_finalize(aggp, cntp, h, wt, b2)
    return h.astype(jnp.float64)


# speedup vs baseline: 675.8326x; 2.3414x over previous
import functools

def _z(i):
    return i * 0

import jax
import jax.numpy as jnp
from jax import lax
from jax.experimental import pallas as pl
from jax.experimental.pallas import tpu as pltpu
from jax.experimental.pallas import tpu_sc as plsc

N = 10000
E = 160000
WD = 32
EF = 16

NC = 2
NS = 16
NW = NC * NS
EPW = E // NW
CHUNK = 1000
NCHUNK = EPW // CHUNK
RPT = N // NS

def _gather_body(x_hbm, src_hbm, out_hbm, idx_v, rows_v, sem):
    wid = lax.axis_index("c") * NS + lax.axis_index("s")
    base = wid * EPW
    for c in range(NCHUNK):
        off = base + c * CHUNK
        pltpu.sync_copy(src_hbm.at[pl.ds(off, CHUNK)], idx_v)
        pltpu.async_copy(x_hbm.at[idx_v], rows_v, sem).wait()
        pltpu.sync_copy(rows_v, out_hbm.at[pl.ds(off, CHUNK)])


@functools.lru_cache(maxsize=None)
def _sc_kernels():
    mesh = plsc.VectorSubcoreMesh(core_axis_name="c", subcore_axis_name="s")
    cp = pltpu.CompilerParams(use_tc_tiling_on_sc=False)
    gather = pl.kernel(
        _gather_body,
        mesh=mesh,
        compiler_params=cp,
        out_type=jax.ShapeDtypeStruct((E, WD), jnp.float32),
        scratch_types=[
            pltpu.VMEM((CHUNK,), jnp.int32),
            pltpu.VMEM((CHUNK, WD), jnp.float32),
            pltpu.SemaphoreType.DMA,
        ],
    )
    scatter_cnt = pl.kernel(
        functools.partial(_scatter_body, True),
        mesh=mesh,
        compiler_params=cp,
        out_type=(
            jax.ShapeDtypeStruct((NC * N, WD), jnp.float32),
            jax.ShapeDtypeStruct((NC * N, EF), jnp.float32),
        ),
        scratch_types=[
            pltpu.VMEM((CHUNK,), jnp.int32),
            pltpu.VMEM((CHUNK, WD), jnp.float32),
            pltpu.VMEM((CHUNK, EF), jnp.float32),
            pltpu.VMEM_SHARED((N, WD), jnp.float32),
            pltpu.VMEM_SHARED((N, EF), jnp.float32),
        ],
    )
    scatter = pl.kernel(
        functools.partial(_scatter_body, False),
        mesh=mesh,
        compiler_params=cp,
        out_type=jax.ShapeDtypeStruct((NC * N, WD), jnp.float32),
        scratch_types=[
            pltpu.VMEM((CHUNK,), jnp.int32),
            pltpu.VMEM((CHUNK, WD), jnp.float32),
            pltpu.VMEM_SHARED((N, WD), jnp.float32),
        ],
    )
    return gather, scatter_cnt, scatter


def _scatter_body(with_cnt, msg_hbm, dst_hbm, z32_hbm, *rest):
    if with_cnt:
        (z16_hbm, ones_hbm, agg_hbm, cnt_hbm,
         idx_v, rows_v, ones_v, agg_sh, cnt_sh) = rest
    else:
        agg_hbm, idx_v, rows_v, agg_sh = rest
    cid = lax.axis_index("c")
    sid = lax.axis_index("s")
    wid = cid * NS + sid
    pltpu.sync_copy(z32_hbm.at[pl.ds(sid * RPT, RPT)],
                    agg_sh.at[pl.ds(sid * RPT, RPT)])
    if with_cnt:
        pltpu.sync_copy(z16_hbm.at[pl.ds(sid * RPT, RPT)],
                        cnt_sh.at[pl.ds(sid * RPT, RPT)])
        pltpu.sync_copy(ones_hbm, ones_v)
    plsc.subcore_barrier()
    base = wid * EPW
    for c in range(NCHUNK):
        off = base + c * CHUNK
        pltpu.sync_copy(dst_hbm.at[pl.ds(off, CHUNK)], idx_v)
        pltpu.sync_copy(msg_hbm.at[pl.ds(off, CHUNK)], rows_v)
        pltpu.sync_copy(rows_v, agg_sh.at[idx_v], add=True)
        if with_cnt:
            pltpu.sync_copy(ones_v, cnt_sh.at[idx_v], add=True)
    plsc.subcore_barrier()
    pltpu.sync_copy(agg_sh.at[pl.ds(sid * RPT, RPT)],
                    agg_hbm.at[pl.ds(cid * N + sid * RPT, RPT)])
    if with_cnt:
        pltpu.sync_copy(cnt_sh.at[pl.ds(sid * RPT, RPT)],
                        cnt_hbm.at[pl.ds(cid * N + sid * RPT, RPT)])


BE = 4000


def _edge_body(xs_ref, ea_ref, rt_ref, tt_ref, m2_ref, dbt_ref, msg_ref):
    xs = xs_ref[...]
    arep = jnp.dot(ea_ref[...], rt_ref[...], preferred_element_type=jnp.float32)
    xst = jnp.dot(xs, tt_ref[...], preferred_element_type=jnp.float32)
    z = arep * xst
    msg_ref[...] = (
        jnp.dot(z, m2_ref[...], preferred_element_type=jnp.float32)
        + jnp.dot(xs, dbt_ref[...], preferred_element_type=jnp.float32))


def _tc_edge(xs, ea, rt, tt, m2, dbt):
    return pl.pallas_call(
        _edge_body,
        grid=(E // BE,),
        in_specs=[
            pl.BlockSpec((BE, WD), lambda i: (i, _z(i))),
            pl.BlockSpec((BE, EF), lambda i: (i, _z(i))),
            pl.BlockSpec((EF, EF * WD), lambda i: (_z(i), _z(i))),
            pl.BlockSpec((WD, EF * WD), lambda i: (_z(i), _z(i))),
            pl.BlockSpec((EF * WD, WD), lambda i: (_z(i), _z(i))),
            pl.BlockSpec((WD, WD), lambda i: (_z(i), _z(i))),
        ],
        out_specs=pl.BlockSpec((BE, WD), lambda i: (i, _z(i))),
        out_shape=jax.ShapeDtypeStruct((E, WD), jnp.float32),
    )(xs, ea, rt, tt, m2, dbt)


BN = 2000


def _fin_body(a0_ref, a1_ref, c0_ref, c1_ref, x_ref, wt_ref, b_ref, out_ref):
    agg = a0_ref[...] + a1_ref[...]
    cnt = c0_ref[:, 0:1] + c1_ref[:, 0:1]
    mean = agg / jnp.maximum(cnt, 1.0)
    lin = jnp.dot(x_ref[...], wt_ref[...], preferred_element_type=jnp.float32,
                  precision=lax.Precision.HIGHEST)
    out_ref[...] = jnp.tanh(mean + lin + b_ref[...])


def _tc_finalize(aggp, cntp, x, wt, b2):
    nb = N // BN
    return pl.pallas_call(
        _fin_body,
        grid=(nb,),
        in_specs=[
            pl.BlockSpec((BN, WD), lambda i: (i, _z(i))),
            pl.BlockSpec((BN, WD), lambda i: (i + nb, _z(i))),
            pl.BlockSpec((BN, EF), lambda i: (i, _z(i))),
            pl.BlockSpec((BN, EF), lambda i: (i + nb, _z(i))),
            pl.BlockSpec((BN, WD), lambda i: (i, _z(i))),
            pl.BlockSpec((WD, WD), lambda i: (_z(i), _z(i))),
            pl.BlockSpec((1, WD), lambda i: (_z(i), _z(i))),
        ],
        out_specs=pl.BlockSpec((BN, WD), lambda i: (i, _z(i))),
        out_shape=jax.ShapeDtypeStruct((N, WD), jnp.float32),
    )(aggp, aggp, cntp, cntp, x, wt, b2)


def kernel(x, edge_index, edge_attr, dW0, db0, W0, b0, dW1, db1, W1, b1):
    src = edge_index[0].astype(jnp.int32)
    dst = edge_index[1].astype(jnp.int32)
    ea = edge_attr.astype(jnp.float32)
    h = x.astype(jnp.float32)
    z32 = jnp.zeros((N, WD), jnp.float32)
    z16 = jnp.zeros((N, EF), jnp.float32)
    ones16 = jnp.ones((CHUNK, EF), jnp.float32)

    rt = jnp.kron(jnp.eye(EF, dtype=jnp.float32),
                  jnp.ones((1, WD), jnp.float32))
    tt = jnp.tile(jnp.eye(WD, dtype=jnp.float32), (1, EF))

    cntp = None
    for dW, db, W, b in ((dW0, db0, W0, b0), (dW1, db1, W1, b1)):
        m2 = jnp.transpose(dW.astype(jnp.float32).reshape(EF, WD, WD),
                           (0, 2, 1)).reshape(EF * WD, WD)
        dbt = db.astype(jnp.float32).reshape(WD, WD).T
        wt = W.astype(jnp.float32).T
        b2 = b.astype(jnp.float32).reshape(1, WD)

        sc_gather, sc_scatter_cnt, sc_scatter = _sc_kernels()
        xs = sc_gather(h, src)
        msg = _tc_edge(xs, ea, rt, tt, m2, dbt)
        if cntp is None:
            aggp, cntp = sc_scatter_cnt(msg, dst, z32, z16, ones16)
        else:
            aggp = sc_scatter(msg, dst, z32)
        h = _tc_finalize(aggp, cntp, h, wt, b2)
    return h.astype(jnp.float64)


# speedup vs baseline: 913.0978x; 1.3511x over previous
import functools

import jax
import jax.numpy as jnp
from jax import lax
from jax.experimental import pallas as pl
from jax.experimental.pallas import tpu as pltpu
from jax.experimental.pallas import tpu_sc as plsc


def _z(i):
    return i * 0


N = 10000
E = 160000
WD = 32
EF = 16
LW = 128
N2 = 10240

NC = 2
NS = 16
NW = NC * NS
EPW = E // NW
CHUNK = 1000
NCHUNK = EPW // CHUNK
RPT = N2 // NS


HW = 48


def _gather_body(x_hbm, src_hbm, out_hbm, idx_v, rows_v, sem):
    wid = lax.axis_index("c") * NS + lax.axis_index("s")
    base = wid * EPW
    for c in range(NCHUNK):
        off = base + c * CHUNK
        pltpu.sync_copy(src_hbm.at[pl.ds(off, CHUNK)], idx_v)
        pltpu.async_copy(x_hbm.at[idx_v], rows_v, sem).wait()
        pltpu.sync_copy(rows_v, out_hbm.at[pl.ds(off, CHUNK), pl.ds(0, WD)])


def _scatter_body(msg_hbm, dst_hbm, zz_hbm, agg_hbm, idx_v, rows_v, agg_sh):
    cid = lax.axis_index("c")
    sid = lax.axis_index("s")
    wid = cid * NS + sid
    pltpu.sync_copy(zz_hbm, agg_sh.at[pl.ds(sid * RPT, RPT)])
    plsc.subcore_barrier()
    base = wid * EPW
    for c in range(NCHUNK):
        off = base + c * CHUNK
        pltpu.sync_copy(dst_hbm.at[pl.ds(off, CHUNK)], idx_v)
        pltpu.sync_copy(msg_hbm.at[pl.ds(off, CHUNK), pl.ds(0, HW)], rows_v)
        pltpu.sync_copy(rows_v, agg_sh.at[idx_v], add=True)
    plsc.subcore_barrier()
    pltpu.sync_copy(agg_sh.at[pl.ds(sid * RPT, RPT)],
                    agg_hbm.at[pl.ds(cid * N2 + sid * RPT, RPT), pl.ds(0, HW)])


@functools.lru_cache(maxsize=None)
def _sc_kernels():
    mesh = plsc.VectorSubcoreMesh(core_axis_name="c", subcore_axis_name="s")
    cp = pltpu.CompilerParams(use_tc_tiling_on_sc=False)
    gather = pl.kernel(
        _gather_body,
        mesh=mesh,
        compiler_params=cp,
        out_type=jax.ShapeDtypeStruct((E, LW), jnp.float32),
        scratch_types=[
            pltpu.VMEM((CHUNK,), jnp.int32),
            pltpu.VMEM((CHUNK, WD), jnp.float32),
            pltpu.SemaphoreType.DMA,
        ],
    )
    scatter = pl.kernel(
        _scatter_body,
        mesh=mesh,
        compiler_params=cp,
        out_type=jax.ShapeDtypeStruct((NC * N2, LW), jnp.float32),
        scratch_types=[
            pltpu.VMEM((CHUNK,), jnp.int32),
            pltpu.VMEM((CHUNK, HW), jnp.float32),
            pltpu.VMEM_SHARED((N2, HW), jnp.float32),
        ],
    )
    return gather, scatter


BE = 4000


def _edge_body(xs_ref, ea_ref, rt_ref, tt_ref, m2_ref, dbt_ref, cv_ref,
               msg_ref):
    xs = xs_ref[:, :WD]
    arep = jnp.dot(ea_ref[...], rt_ref[...], preferred_element_type=jnp.float32)
    xst = jnp.dot(xs, tt_ref[...], preferred_element_type=jnp.float32)
    z = arep * xst
    msg_ref[...] = (
        jnp.dot(z, m2_ref[...], preferred_element_type=jnp.float32)
        + jnp.dot(xs, dbt_ref[...], preferred_element_type=jnp.float32)
        + cv_ref[...])


def _tc_edge(xs, ea, rt, tt, m2, dbt, cv):
    return pl.pallas_call(
        _edge_body,
        grid=(E // BE,),
        in_specs=[
            pl.BlockSpec((BE, LW), lambda i: (i, _z(i))),
            pl.BlockSpec((BE, EF), lambda i: (i, _z(i))),
            pl.BlockSpec((EF, EF * WD), lambda i: (_z(i), _z(i))),
            pl.BlockSpec((WD, EF * WD), lambda i: (_z(i), _z(i))),
            pl.BlockSpec((EF * WD, LW), lambda i: (_z(i), _z(i))),
            pl.BlockSpec((WD, LW), lambda i: (_z(i), _z(i))),
            pl.BlockSpec((1, LW), lambda i: (_z(i), _z(i))),
        ],
        out_specs=pl.BlockSpec((BE, LW), lambda i: (i, _z(i))),
        out_shape=jax.ShapeDtypeStruct((E, LW), jnp.float32),
    )(xs, ea, rt, tt, m2, dbt, cv)


BN = 1024


def _fin_body(a0_ref, a1_ref, x_ref, wt_ref, b_ref, out_ref):
    agg = a0_ref[...] + a1_ref[...]
    cnt = agg[:, WD:WD + 1]
    mean = agg / jnp.maximum(cnt, 1.0)
    lin = jnp.dot(x_ref[...], wt_ref[...], preferred_element_type=jnp.float32,
                  precision=lax.Precision.HIGHEST)
    out_ref[...] = jnp.tanh(mean + lin + b_ref[...])


def _tc_finalize(aggp, x, wt, b2):
    nb = N2 // BN
    return pl.pallas_call(
        _fin_body,
        grid=(nb,),
        in_specs=[
            pl.BlockSpec((BN, LW), lambda i: (i, _z(i))),
            pl.BlockSpec((BN, LW), lambda i: (i + nb, _z(i))),
            pl.BlockSpec((BN, LW), lambda i: (i, _z(i))),
            pl.BlockSpec((LW, LW), lambda i: (_z(i), _z(i))),
            pl.BlockSpec((1, LW), lambda i: (_z(i), _z(i))),
        ],
        out_specs=pl.BlockSpec((BN, LW), lambda i: (i, _z(i))),
        out_shape=jax.ShapeDtypeStruct((N2, LW), jnp.float32),
    )(aggp, aggp, x, wt, b2)


def kernel(x, edge_index, edge_attr, dW0, db0, W0, b0, dW1, db1, W1, b1):
    f32 = jnp.float32
    src = edge_index[0].astype(jnp.int32)
    dst = edge_index[1].astype(jnp.int32)
    ea = edge_attr.astype(f32)
    h = jnp.zeros((N2, LW), f32).at[:N, :WD].set(x.astype(f32))
    zz = jnp.zeros((RPT, HW), f32)

    rt = jnp.kron(jnp.eye(EF, dtype=f32), jnp.ones((1, WD), f32))
    tt = jnp.tile(jnp.eye(WD, dtype=f32), (1, EF))
    cv = jnp.zeros((1, LW), f32).at[0, WD].set(1.0)

    for dW, db, W, b in ((dW0, db0, W0, b0), (dW1, db1, W1, b1)):
        m2 = jnp.zeros((EF * WD, LW), f32).at[:, :WD].set(
            jnp.transpose(dW.astype(f32).reshape(EF, WD, WD),
                          (0, 2, 1)).reshape(EF * WD, WD))
        dbt = jnp.zeros((WD, LW), f32).at[:, :WD].set(
            db.astype(f32).reshape(WD, WD).T)
        wt = jnp.zeros((LW, LW), f32).at[:WD, :WD].set(W.astype(f32).T)
        b2 = jnp.zeros((1, LW), f32).at[0, :WD].set(b.astype(f32))

        sc_gather, sc_scatter = _sc_kernels()
        xs = sc_gather(h[:, :WD], src)
        msg = _tc_edge(xs, ea, rt, tt, m2, dbt, cv)
        aggp = sc_scatter(msg, dst, zz)
        h = _tc_finalize(aggp, h, wt, b2)
    return h[:N, :WD].astype(jnp.float64)


# speedup vs baseline: 975.2624x; 1.0681x over previous
import functools

import jax
import jax.numpy as jnp
from jax import lax
from jax.experimental import pallas as pl
from jax.experimental.pallas import tpu as pltpu
from jax.experimental.pallas import tpu_sc as plsc


def _z(i):
    return i * 0


N = 10000
E = 160000
WD = 32
EF = 16
LW = 128
N2 = 10240

NC = 2
NS = 16
NW = NC * NS
EPW = E // NW
CHUNK = 1000
NCHUNK = EPW // CHUNK
RPT = N2 // NS


HW = 48


def _gather_body(x_hbm, src_hbm, out_hbm, idx_v, rows_v, sem):
    wid = lax.axis_index("c") * NS + lax.axis_index("s")
    base = wid * EPW
    for c in range(NCHUNK):
        off = base + c * CHUNK
        pltpu.sync_copy(src_hbm.at[pl.ds(off, CHUNK)], idx_v)
        pltpu.async_copy(x_hbm.at[idx_v], rows_v, sem).wait()
        pltpu.sync_copy(rows_v, out_hbm.at[pl.ds(off, CHUNK), pl.ds(0, WD)])


def _scatter_body(msg_hbm, dst_hbm, zz_hbm, agg_hbm, idx_v, rows_v, agg_sh):
    cid = lax.axis_index("c")
    sid = lax.axis_index("s")
    wid = cid * NS + sid
    pltpu.sync_copy(zz_hbm, agg_sh.at[pl.ds(sid * RPT, RPT)])
    plsc.subcore_barrier()
    base = wid * EPW
    for c in range(NCHUNK):
        off = base + c * CHUNK
        pltpu.sync_copy(dst_hbm.at[pl.ds(off, CHUNK)], idx_v)
        pltpu.sync_copy(msg_hbm.at[pl.ds(off, CHUNK), pl.ds(0, HW)], rows_v)
        pltpu.sync_copy(rows_v, agg_sh.at[idx_v], add=True)
    plsc.subcore_barrier()
    pltpu.sync_copy(agg_sh.at[pl.ds(sid * RPT, RPT)],
                    agg_hbm.at[pl.ds(cid * N2 + sid * RPT, RPT), pl.ds(0, HW)])


@functools.lru_cache(maxsize=None)
def _sc_kernels():
    mesh = plsc.VectorSubcoreMesh(core_axis_name="c", subcore_axis_name="s")
    cp = pltpu.CompilerParams(use_tc_tiling_on_sc=False)
    gather = pl.kernel(
        _gather_body,
        mesh=mesh,
        compiler_params=cp,
        out_type=jax.ShapeDtypeStruct((E, LW), jnp.float32),
        scratch_types=[
            pltpu.VMEM((CHUNK,), jnp.int32),
            pltpu.VMEM((CHUNK, WD), jnp.float32),
            pltpu.SemaphoreType.DMA,
        ],
    )
    scatter = pl.kernel(
        _scatter_body,
        mesh=mesh,
        compiler_params=cp,
        out_type=jax.ShapeDtypeStruct((NC * N2, LW), jnp.float32),
        scratch_types=[
            pltpu.VMEM((CHUNK,), jnp.int32),
            pltpu.VMEM((CHUNK, HW), jnp.float32),
            pltpu.VMEM_SHARED((N2, HW), jnp.float32),
        ],
    )
    return gather, scatter


BE = 6400


def _edge_body(xs_ref, eat_ref, rt_ref, tt_ref, m2_ref, dbt_ref, cv_ref,
               msg_ref):
    bf = jnp.bfloat16
    xs = xs_ref[:, :WD].astype(bf)
    arep = lax.dot_general(eat_ref[...].astype(bf), rt_ref[...],
                           (((0,), (0,)), ((), ())),
                           preferred_element_type=jnp.float32).astype(bf)
    xst = jnp.dot(xs, tt_ref[...],
                  preferred_element_type=jnp.float32).astype(bf)
    z = arep * xst
    m33 = (jnp.dot(z, m2_ref[...], preferred_element_type=jnp.float32)
           + jnp.dot(xs, dbt_ref[...], preferred_element_type=jnp.float32)
           + cv_ref[...])
    msg_ref[...] = jnp.concatenate(
        [m33, jnp.zeros((BE, LW - 2 * WD), jnp.float32)], axis=1)


def _tc_edge(xs, eat, rt, tt, m2, dbt, cv):
    return pl.pallas_call(
        _edge_body,
        grid=(E // BE,),
        in_specs=[
            pl.BlockSpec((BE, LW), lambda i: (i, _z(i))),
            pl.BlockSpec((EF, BE), lambda i: (_z(i), i)),
            pl.BlockSpec((EF, EF * WD), lambda i: (_z(i), _z(i))),
            pl.BlockSpec((WD, EF * WD), lambda i: (_z(i), _z(i))),
            pl.BlockSpec((EF * WD, 2 * WD), lambda i: (_z(i), _z(i))),
            pl.BlockSpec((WD, 2 * WD), lambda i: (_z(i), _z(i))),
            pl.BlockSpec((1, 2 * WD), lambda i: (_z(i), _z(i))),
        ],
        out_specs=pl.BlockSpec((BE, LW), lambda i: (i, _z(i))),
        out_shape=jax.ShapeDtypeStruct((E, LW), jnp.float32),
    )(xs, eat, rt, tt, m2, dbt, cv)


BN = 1024


def _fin_body(a0_ref, a1_ref, x_ref, wt_ref, b_ref, out_ref):
    agg = a0_ref[...] + a1_ref[...]
    cnt = agg[:, WD:WD + 1]
    mean = agg / jnp.maximum(cnt, 1.0)
    lin = jnp.dot(x_ref[...], wt_ref[...], preferred_element_type=jnp.float32,
                  precision=lax.Precision.HIGHEST)
    out_ref[...] = jnp.tanh(mean + lin + b_ref[...])


def _tc_finalize(aggp, x, wt, b2):
    nb = N2 // BN
    return pl.pallas_call(
        _fin_body,
        grid=(nb,),
        in_specs=[
            pl.BlockSpec((BN, LW), lambda i: (i, _z(i))),
            pl.BlockSpec((BN, LW), lambda i: (i + nb, _z(i))),
            pl.BlockSpec((BN, LW), lambda i: (i, _z(i))),
            pl.BlockSpec((LW, LW), lambda i: (_z(i), _z(i))),
            pl.BlockSpec((1, LW), lambda i: (_z(i), _z(i))),
        ],
        out_specs=pl.BlockSpec((BN, LW), lambda i: (i, _z(i))),
        out_shape=jax.ShapeDtypeStruct((N2, LW), jnp.float32),
    )(aggp, aggp, x, wt, b2)


def kernel(x, edge_index, edge_attr, dW0, db0, W0, b0, dW1, db1, W1, b1):
    f32 = jnp.float32
    src = edge_index[0].astype(jnp.int32)
    dst = edge_index[1].astype(jnp.int32)
    eat = edge_attr.astype(f32).T
    h = jnp.zeros((N2, LW), f32).at[:N, :WD].set(x.astype(f32))
    zz = jnp.zeros((RPT, HW), f32)

    bf = jnp.bfloat16
    rt = jnp.kron(jnp.eye(EF, dtype=bf), jnp.ones((1, WD), bf))
    tt = jnp.tile(jnp.eye(WD, dtype=bf), (1, EF))
    cv = jnp.zeros((1, 2 * WD), f32).at[0, WD].set(1.0)

    for dW, db, W, b in ((dW0, db0, W0, b0), (dW1, db1, W1, b1)):
        m2 = jnp.zeros((EF * WD, 2 * WD), bf).at[:, :WD].set(
            jnp.transpose(dW.astype(bf).reshape(EF, WD, WD),
                          (0, 2, 1)).reshape(EF * WD, WD))
        dbt = jnp.zeros((WD, 2 * WD), bf).at[:, :WD].set(
            db.astype(bf).reshape(WD, WD).T)
        wt = jnp.zeros((LW, LW), f32).at[:WD, :WD].set(W.astype(f32).T)
        b2 = jnp.zeros((1, LW), f32).at[0, :WD].set(b.astype(f32))

        sc_gather, sc_scatter = _sc_kernels()
        xs = sc_gather(h[:, :WD], src)
        msg = _tc_edge(xs, eat, rt, tt, m2, dbt, cv)
        aggp = sc_scatter(msg, dst, zz)
        h = _tc_finalize(aggp, h, wt, b2)
    return h[:N, :WD].astype(jnp.float64)
